# Initial kernel scaffold; baseline (speedup 1.0000x reference)
import jax
import jax.numpy as jnp
from jax.experimental import pallas as pl


def kernel(x, w1, b):
    raise NotImplementedError("write your pallas kernel here")


# speedup vs baseline: 1.4592x; 1.4592x over previous
import jax
import jax.numpy as jnp
from jax.experimental import pallas as pl
from jax.experimental.pallas import tpu as pltpu


def _gram_kernel(w_ref, g_ref):
    w = w_ref[...].astype(jnp.bfloat16)
    g = jax.lax.dot_general(
        w, w, (((1,), (1,)), ((), ())),
        preferred_element_type=jnp.float32,
    )
    g_ref[...] = g.astype(jnp.bfloat16)


def _rows_kernel(x_ref, g_ref, b_ref, o_ref):
    x = x_ref[...].astype(jnp.bfloat16)
    y = jnp.dot(x, g_ref[...], preferred_element_type=jnp.float32)
    o_ref[...] = jnp.maximum(y + b_ref[...], 0.0).astype(o_ref.dtype)


def _round_up(v, m):
    return ((v + m - 1) // m) * m


def kernel(x, w1, b):
    n_data, n_feat = x.shape
    nf_w, n_hidden = w1.shape
    assert nf_w == n_feat

    b2d = b.reshape(1, n_feat)

    gram_cost = pl.CostEstimate(
        flops=2 * n_feat * n_feat * n_hidden,
        transcendentals=0,
        bytes_accessed=n_feat * n_hidden * 4 + n_feat * n_feat * 2,
    )
    g = pl.pallas_call(
        _gram_kernel,
        out_shape=jax.ShapeDtypeStruct((n_feat, n_feat), jnp.bfloat16),
        in_specs=[pl.BlockSpec(memory_space=pltpu.MemorySpace.VMEM)],
        out_specs=pl.BlockSpec(memory_space=pltpu.MemorySpace.VMEM),
        cost_estimate=gram_cost,
    )(w1)

    tm = min(1024, _round_up(n_data, 8))
    grid = (pl.cdiv(n_data, tm),)

    main_cost = pl.CostEstimate(
        flops=2 * n_data * n_feat * n_feat,
        transcendentals=0,
        bytes_accessed=2 * n_data * n_feat * 4 + n_feat * n_feat * 2,
    )
    return pl.pallas_call(
        _rows_kernel,
        out_shape=jax.ShapeDtypeStruct((n_data, n_feat), x.dtype),
        grid=grid,
        in_specs=[
            pl.BlockSpec((tm, n_feat), lambda i: (i, 0)),
            pl.BlockSpec((n_feat, n_feat), lambda i: (0, 0)),
            pl.BlockSpec((1, n_feat), lambda i: (0, 0)),
        ],
        out_specs=pl.BlockSpec((tm, n_feat), lambda i: (i, 0)),
        cost_estimate=main_cost,
        compiler_params=pltpu.CompilerParams(
            dimension_semantics=("parallel",),
            vmem_limit_bytes=48 * 1024 * 1024,
        ),
    )(x, g, b2d)


# speedup vs baseline: 1.5227x; 1.0436x over previous
import jax
import jax.numpy as jnp
from jax.experimental import pallas as pl
from jax.experimental.pallas import tpu as pltpu


def _gram_kernel(w_ref, g_ref):
    w = w_ref[...].astype(jnp.bfloat16)
    g = jax.lax.dot_general(
        w, w, (((1,), (1,)), ((), ())),
        preferred_element_type=jnp.float32,
    )
    g_ref[...] = g.astype(jnp.bfloat16)


def _rows_kernel(x_ref, g_ref, b_ref, o_ref):
    x = x_ref[...].astype(jnp.bfloat16)
    y = jnp.dot(x, g_ref[...], preferred_element_type=jnp.float32)
    o_ref[...] = jnp.maximum(y + b_ref[...], 0.0).astype(o_ref.dtype)


def _round_up(v, m):
    return ((v + m - 1) // m) * m


def kernel(x, w1, b):
    n_data, n_feat = x.shape
    nf_w, n_hidden = w1.shape
    assert nf_w == n_feat

    b2d = b.reshape(1, n_feat)

    gram_cost = pl.CostEstimate(
        flops=2 * n_feat * n_feat * n_hidden,
        transcendentals=0,
        bytes_accessed=n_feat * n_hidden * 4 + n_feat * n_feat * 2,
    )
    g = pl.pallas_call(
        _gram_kernel,
        out_shape=jax.ShapeDtypeStruct((n_feat, n_feat), jnp.bfloat16),
        in_specs=[pl.BlockSpec(memory_space=pltpu.MemorySpace.VMEM)],
        out_specs=pl.BlockSpec(memory_space=pltpu.MemorySpace.VMEM),
        cost_estimate=gram_cost,
    )(w1)

    tm = min(2048, _round_up(n_data, 8))
    grid = (pl.cdiv(n_data, tm),)

    main_cost = pl.CostEstimate(
        flops=2 * n_data * n_feat * n_feat,
        transcendentals=0,
        bytes_accessed=2 * n_data * n_feat * 4 + n_feat * n_feat * 2,
    )
    return pl.pallas_call(
        _rows_kernel,
        out_shape=jax.ShapeDtypeStruct((n_data, n_feat), x.dtype),
        grid=grid,
        in_specs=[
            pl.BlockSpec((tm, n_feat), lambda i: (i, 0)),
            pl.BlockSpec((n_feat, n_feat), lambda i: (0, 0)),
            pl.BlockSpec((1, n_feat), lambda i: (0, 0)),
        ],
        out_specs=pl.BlockSpec((tm, n_feat), lambda i: (i, 0)),
        cost_estimate=main_cost,
        compiler_params=pltpu.CompilerParams(
            dimension_semantics=("parallel",),
            vmem_limit_bytes=48 * 1024 * 1024,
        ),
    )(x, g, b2d)
